# baseline (device time: 64018 ns/iter reference)
import jax
import jax.numpy as jnp
from jax import lax
from jax.experimental import pallas as pl
from jax.experimental.pallas import tpu as pltpu

N_DEV = 4
B_SHARD = 64
B = 256
D = 512
BF16 = jnp.bfloat16
F32 = jnp.float32

_sem_signal = getattr(pl, "semaphore_signal", None) or pltpu.semaphore_signal
_sem_wait = getattr(pl, "semaphore_wait", None) or pltpu.semaphore_wait
_DeviceIdType = getattr(pl, "DeviceIdType", None) or pltpu.DeviceIdType

_N_SEMS = 3 + 3 * 3


def kernel(x, Win0, Wout0, Win1, Wout1, Win2, Wout2):
    def body(
        x_ref,
        win0_ref,
        wout0_ref,
        win1_ref,
        wout1_ref,
        win2_ref,
        wout2_ref,
        out_ref,
        xfull_ref,
        ag_ref,
        ar_ref,
        send_sems,
        recv_sems,
    ):
        my = lax.axis_index("i")
        left = (my + N_DEV - 1) % N_DEV
        right = (my + 1) % N_DEV

        barrier_sem = pltpu.get_barrier_semaphore()
        for nbr in (left, right):
            _sem_signal(
                barrier_sem,
                inc=1,
                device_id=(nbr,),
                device_id_type=_DeviceIdType.MESH,
            )
        _sem_wait(barrier_sem, 2)

        myx = x_ref[...].astype(BF16)
        xfull_ref[pl.ds(my * B_SHARD, B_SHARD), :] = myx
        ag_ref[0, :, :] = myx
        for h in range(N_DEV - 1):
            rdma = pltpu.make_async_remote_copy(
                src_ref=ag_ref.at[h],
                dst_ref=ag_ref.at[h + 1],
                send_sem=send_sems.at[h],
                recv_sem=recv_sems.at[h],
                device_id=(right,),
                device_id_type=_DeviceIdType.MESH,
            )
            rdma.start()
            rdma.wait()
            origin = (my + N_DEV - 1 - h) % N_DEV
            xfull_ref[pl.ds(origin * B_SHARD, B_SHARD), :] = ag_ref[h + 1, :, :]

        weights = [
            (win0_ref, wout0_ref),
            (win1_ref, wout1_ref),
            (win2_ref, wout2_ref),
        ]
        xf = xfull_ref[...]
        for L, (win_ref, wout_ref) in enumerate(weights):
            hidden = jnp.dot(
                xf, win_ref[...].astype(BF16), preferred_element_type=F32
            )
            hidden = jnp.maximum(hidden, 0.0).astype(BF16)
            partial = jnp.dot(
                hidden, wout_ref[...].astype(BF16), preferred_element_type=F32
            )
            base = 4 * L
            ar_ref[base, :, :] = partial.astype(BF16)
            acc = partial
            for h in range(N_DEV - 1):
                sem = 3 + 3 * L + h
                rdma = pltpu.make_async_remote_copy(
                    src_ref=ar_ref.at[base + h],
                    dst_ref=ar_ref.at[base + h + 1],
                    send_sem=send_sems.at[sem],
                    recv_sem=recv_sems.at[sem],
                    device_id=(right,),
                    device_id_type=_DeviceIdType.MESH,
                )
                rdma.start()
                rdma.wait()
                acc = acc + ar_ref[base + h + 1, :, :].astype(F32)
            if L < 2:
                xf = acc.astype(BF16)
            else:
                out_ref[...] = acc

    return pl.pallas_call(
        body,
        out_shape=jax.ShapeDtypeStruct((B, D), F32),
        in_specs=[pl.BlockSpec(memory_space=pltpu.VMEM)] * 7,
        out_specs=pl.BlockSpec(memory_space=pltpu.VMEM),
        scratch_shapes=[
            pltpu.VMEM((B, D), BF16),
            pltpu.VMEM((N_DEV, B_SHARD, D), BF16),
            pltpu.VMEM((3 * N_DEV, B, D), BF16),
            pltpu.SemaphoreType.DMA((_N_SEMS,)),
            pltpu.SemaphoreType.DMA((_N_SEMS,)),
        ],
        compiler_params=pltpu.CompilerParams(collective_id=0),
    )(x, Win0, Wout0, Win1, Wout1, Win2, Wout2)


# device time: 44464 ns/iter; 1.4398x vs baseline; 1.4398x over previous
import jax
import jax.numpy as jnp
from jax import lax
from jax.experimental import pallas as pl
from jax.experimental.pallas import tpu as pltpu

N_DEV = 4
N_PEER = 3
B_SHARD = 64
B = 256
D = 512
N_LAYER = 3
N_PHASE = 1 + N_LAYER
BF16 = jnp.bfloat16
F32 = jnp.float32

_sem_signal = getattr(pl, "semaphore_signal", None) or pltpu.semaphore_signal
_sem_wait = getattr(pl, "semaphore_wait", None) or pltpu.semaphore_wait
_DeviceIdType = getattr(pl, "DeviceIdType", None) or pltpu.DeviceIdType


def kernel(x, Win0, Wout0, Win1, Wout1, Win2, Wout2):
    def body(
        x_ref,
        win0_ref,
        wout0_ref,
        win1_ref,
        wout1_ref,
        win2_ref,
        wout2_ref,
        out_ref,
        xfull_ref,
        agx_ref,
        psend_ref,
        arbuf_ref,
        send_sems,
        recv_sems,
    ):
        my = lax.axis_index("i")
        left = (my + N_DEV - 1) % N_DEV
        right = (my + 1) % N_DEV
        opp = (my + 2) % N_DEV
        peers = [left, right, opp]

        barrier_sem = pltpu.get_barrier_semaphore()
        for p in peers:
            _sem_signal(
                barrier_sem,
                inc=1,
                device_id=(p,),
                device_id_type=_DeviceIdType.MESH,
            )
        _sem_wait(barrier_sem, N_PEER)

        def exchange_start(phase, src_ref, dst_slots):
            rdmas = []
            for k, p in enumerate(peers):
                rdma = pltpu.make_async_remote_copy(
                    src_ref=src_ref,
                    dst_ref=dst_slots[k],
                    send_sem=send_sems.at[phase, k],
                    recv_sem=recv_sems.at[phase, k],
                    device_id=(p,),
                    device_id_type=_DeviceIdType.MESH,
                )
                rdma.start()
                rdmas.append(rdma)
            return rdmas

        def exchange_finish(rdmas):
            for rdma in rdmas:
                rdma.wait_recv()
            for rdma in rdmas:
                rdma.wait_send()

        myx = x_ref[...].astype(BF16)
        agx_ref[N_PEER, :, :] = myx
        ag = exchange_start(
            0, agx_ref.at[N_PEER], [agx_ref.at[k] for k in range(N_PEER)]
        )
        xfull_ref[pl.ds(my * B_SHARD, B_SHARD), :] = myx
        exchange_finish(ag)
        for k, src_pos in enumerate([right, left, opp]):
            xfull_ref[pl.ds(src_pos * B_SHARD, B_SHARD), :] = agx_ref[k, :, :]

        weights = [
            (win0_ref, wout0_ref),
            (win1_ref, wout1_ref),
            (win2_ref, wout2_ref),
        ]
        xf = xfull_ref[...]
        for L, (win_ref, wout_ref) in enumerate(weights):
            hidden = jnp.dot(
                xf, win_ref[...].astype(BF16), preferred_element_type=F32
            )
            hidden = jnp.maximum(hidden, 0.0).astype(BF16)
            partial = jnp.dot(
                hidden, wout_ref[...].astype(BF16), preferred_element_type=F32
            )
            psend_ref[L, :, :] = partial.astype(BF16)
            ar = exchange_start(
                1 + L,
                psend_ref.at[L],
                [arbuf_ref.at[L, k] for k in range(N_PEER)],
            )
            exchange_finish(ar)
            acc = partial
            for k in range(N_PEER):
                acc = acc + arbuf_ref[L, k, :, :].astype(F32)
            if L < N_LAYER - 1:
                xf = acc.astype(BF16)
            else:
                out_ref[...] = acc

    return pl.pallas_call(
        body,
        out_shape=jax.ShapeDtypeStruct((B, D), F32),
        in_specs=[pl.BlockSpec(memory_space=pltpu.VMEM)] * 7,
        out_specs=pl.BlockSpec(memory_space=pltpu.VMEM),
        scratch_shapes=[
            pltpu.VMEM((B, D), BF16),
            pltpu.VMEM((N_DEV, B_SHARD, D), BF16),
            pltpu.VMEM((N_LAYER, B, D), BF16),
            pltpu.VMEM((N_LAYER, N_PEER, B, D), BF16),
            pltpu.SemaphoreType.DMA((N_PHASE, N_PEER)),
            pltpu.SemaphoreType.DMA((N_PHASE, N_PEER)),
        ],
        compiler_params=pltpu.CompilerParams(collective_id=0),
    )(x, Win0, Wout0, Win1, Wout1, Win2, Wout2)


# device time: 38592 ns/iter; 1.6588x vs baseline; 1.1522x over previous
import jax
import jax.numpy as jnp
from jax import lax
from jax.experimental import pallas as pl
from jax.experimental.pallas import tpu as pltpu

N_DEV = 4
N_PEER = 3
N_BLK = 4
B_SHARD = 64
B = 256
D = 512
N_LAYER = 3
BF16 = jnp.bfloat16
F32 = jnp.float32

_sem_signal = getattr(pl, "semaphore_signal", None) or pltpu.semaphore_signal
_sem_wait = getattr(pl, "semaphore_wait", None) or pltpu.semaphore_wait
_DeviceIdType = getattr(pl, "DeviceIdType", None) or pltpu.DeviceIdType


def kernel(x, Win0, Wout0, Win1, Wout1, Win2, Wout2):
    def body(
        x_ref,
        win0_ref,
        wout0_ref,
        win1_ref,
        wout1_ref,
        win2_ref,
        wout2_ref,
        out_ref,
        xsrc_ref,
        agx_ref,
        xfull_ref,
        psend_ref,
        prec_ref,
        xg_send,
        xg_recv,
        ar_send,
        ar_recv,
    ):
        my = lax.axis_index("i")
        left = (my + N_DEV - 1) % N_DEV
        right = (my + 1) % N_DEV
        opp = (my + 2) % N_DEV
        peers = [left, right, opp]

        barrier_sem = pltpu.get_barrier_semaphore()
        for p in peers:
            _sem_signal(
                barrier_sem,
                inc=1,
                device_id=(p,),
                device_id_type=_DeviceIdType.MESH,
            )
        _sem_wait(barrier_sem, N_PEER)

        myx = x_ref[...].astype(BF16)
        xsrc_ref[...] = myx
        xg = []
        for k, p in enumerate(peers):
            rdma = pltpu.make_async_remote_copy(
                src_ref=xsrc_ref,
                dst_ref=agx_ref.at[k],
                send_sem=xg_send.at[k],
                recv_sem=xg_recv.at[k],
                device_id=(p,),
                device_id_type=_DeviceIdType.MESH,
            )
            rdma.start()
            xg.append(rdma)
        xfull_ref[pl.ds(my * B_SHARD, B_SHARD), :] = myx
        for rdma in xg:
            rdma.wait_recv()
        for k, src_pos in enumerate([right, left, opp]):
            xfull_ref[pl.ds(src_pos * B_SHARD, B_SHARD), :] = agx_ref[k, :, :]
        for rdma in xg:
            rdma.wait_send()

        weights = [
            (win0_ref, wout0_ref),
            (win1_ref, wout1_ref),
            (win2_ref, wout2_ref),
        ]
        own = [None] * N_BLK
        handles = {}
        for L, (win_ref, wout_ref) in enumerate(weights):
            winb = win_ref[...].astype(BF16)
            woutb = wout_ref[...].astype(BF16)
            for j in range(N_BLK):
                if L == 0:
                    xblk = xfull_ref[j * B_SHARD : (j + 1) * B_SHARD, :]
                else:
                    acc = own[j]
                    for k in range(N_PEER):
                        handles[(L - 1, k, j)].wait_recv()
                    for k in range(N_PEER):
                        acc = acc + prec_ref[L - 1, k, j, :, :].astype(F32)
                    xblk = acc.astype(BF16)
                hid = jnp.maximum(
                    jnp.dot(xblk, winb, preferred_element_type=F32), 0.0
                ).astype(BF16)
                pblk = jnp.dot(hid, woutb, preferred_element_type=F32)
                own[j] = pblk
                psend_ref[L, j, :, :] = pblk.astype(BF16)
                for k, p in enumerate(peers):
                    rdma = pltpu.make_async_remote_copy(
                        src_ref=psend_ref.at[L, j],
                        dst_ref=prec_ref.at[L, k, j],
                        send_sem=ar_send.at[L, k, j],
                        recv_sem=ar_recv.at[L, k, j],
                        device_id=(p,),
                        device_id_type=_DeviceIdType.MESH,
                    )
                    rdma.start()
                    handles[(L, k, j)] = rdma
                if L >= 1:
                    for k in range(N_PEER):
                        handles[(L - 1, k, j)].wait_send()

        Lf = N_LAYER - 1
        for j in range(N_BLK):
            acc = own[j]
            for k in range(N_PEER):
                handles[(Lf, k, j)].wait_recv()
            for k in range(N_PEER):
                acc = acc + prec_ref[Lf, k, j, :, :].astype(F32)
            out_ref[j * B_SHARD : (j + 1) * B_SHARD, :] = acc
            for k in range(N_PEER):
                handles[(Lf, k, j)].wait_send()

    return pl.pallas_call(
        body,
        out_shape=jax.ShapeDtypeStruct((B, D), F32),
        in_specs=[pl.BlockSpec(memory_space=pltpu.VMEM)] * 7,
        out_specs=pl.BlockSpec(memory_space=pltpu.VMEM),
        scratch_shapes=[
            pltpu.VMEM((B_SHARD, D), BF16),
            pltpu.VMEM((N_PEER, B_SHARD, D), BF16),
            pltpu.VMEM((B, D), BF16),
            pltpu.VMEM((N_LAYER, N_BLK, B_SHARD, D), BF16),
            pltpu.VMEM((N_LAYER, N_PEER, N_BLK, B_SHARD, D), BF16),
            pltpu.SemaphoreType.DMA((N_PEER,)),
            pltpu.SemaphoreType.DMA((N_PEER,)),
            pltpu.SemaphoreType.DMA((N_LAYER, N_PEER, N_BLK)),
            pltpu.SemaphoreType.DMA((N_LAYER, N_PEER, N_BLK)),
        ],
        compiler_params=pltpu.CompilerParams(collective_id=0),
    )(x, Win0, Wout0, Win1, Wout1, Win2, Wout2)
